# Initial kernel scaffold; baseline (speedup 1.0000x reference)
import functools

import jax
import jax.numpy as jnp
from jax import lax
from jax.experimental import pallas as pl
from jax.experimental.pallas import tpu as pltpu
from jax.experimental.pallas import tpu_sc as plsc

N = 10000
E = 320000
D = 128
H = 128
C = 2
G = 64
EPS = 1e-5

NP = 10240
NC = 2
NS = 16
L = 16
NW = NC * NS
EW = E // NW
K = 80
CH = EW // K
SLAB = NP // NS
HIGHEST = jax.lax.Precision.HIGHEST

_mesh = plsc.VectorSubcoreMesh(
    core_axis_name="c", subcore_axis_name="s", num_cores=NC, num_subcores=NS)


def _sc_deg_body(dst_hbm, out_hbm, didx_v, ones_v, zcol_v, acc_sh):
    core = lax.axis_index("c")
    sub = lax.axis_index("s")
    w = sub * NC + core

    for i in range(K // L):
        ones_v[pl.ds(i * L, L)] = jnp.ones((L,), jnp.float32)

    def _zero(i, _):
        zcol_v[pl.ds(i * L, L)] = jnp.zeros((L,), jnp.float32)
        return 0
    lax.fori_loop(0, SLAB // L, _zero, 0)
    pltpu.sync_copy(zcol_v, acc_sh.at[pl.ds(sub * SLAB, SLAB)])
    plsc.subcore_barrier()

    pltpu.sync_copy(dst_hbm.at[w], didx_v)

    def _chunk(j, _):
        pltpu.sync_copy(ones_v, acc_sh.at[didx_v.at[j]], add=True)
        return 0
    lax.fori_loop(0, CH, _chunk, 0)
    plsc.subcore_barrier()
    pltpu.sync_copy(acc_sh.at[pl.ds(sub * SLAB, SLAB)],
                    out_hbm.at[core, pl.ds(sub * SLAB, SLAB)])


_sc_deg = functools.partial(
    pl.kernel,
    out_type=jax.ShapeDtypeStruct((NC, NP), jnp.float32),
    mesh=_mesh,
    scratch_types=[
        pltpu.VMEM((CH, K), jnp.int32),
        pltpu.VMEM((K,), jnp.float32),
        pltpu.VMEM((SLAB,), jnp.float32),
        pltpu.VMEM_SHARED((NP,), jnp.float32),
    ],
)(_sc_deg_body)


def _sc_agg_body(y_hbm, src_hbm, dst_hbm, out_hbm, sidx_v, didx_v, rows_v,
                 acc_sh):
    core = lax.axis_index("c")
    sub = lax.axis_index("s")
    w = sub * NC + core

    def _zero(i, _):
        rows_v[i // 8, pl.ds((i % 8) * L, L)] = jnp.zeros((L,), jnp.float32)
        return 0
    lax.fori_loop(0, K * (H // L), _zero, 0)
    for t in range(SLAB // K):
        pltpu.sync_copy(rows_v, acc_sh.at[pl.ds(sub * SLAB + t * K, K)])
    plsc.subcore_barrier()

    pltpu.sync_copy(src_hbm.at[w], sidx_v)
    pltpu.sync_copy(dst_hbm.at[w], didx_v)

    def _chunk(j, _):
        pltpu.sync_copy(y_hbm.at[sidx_v.at[j]], rows_v)
        pltpu.sync_copy(rows_v, acc_sh.at[didx_v.at[j]], add=True)
        return 0
    lax.fori_loop(0, CH, _chunk, 0)
    plsc.subcore_barrier()
    pltpu.sync_copy(acc_sh.at[pl.ds(sub * SLAB, SLAB)],
                    out_hbm.at[core, pl.ds(sub * SLAB, SLAB)])


_sc_agg = functools.partial(
    pl.kernel,
    out_type=jax.ShapeDtypeStruct((NC, NP, H), jnp.float32),
    mesh=_mesh,
    scratch_types=[
        pltpu.VMEM((CH, K), jnp.int32),
        pltpu.VMEM((CH, K), jnp.int32),
        pltpu.VMEM((K, H), jnp.float32),
        pltpu.VMEM_SHARED((NP, H), jnp.float32),
    ],
)(_sc_agg_body)


def _tc_pre_body(x_ref, w1_ref, d0_ref, d1_ref, y1_ref, dinv_ref):
    deg = d0_ref[...] + d1_ref[...] + 1.0
    dinv = 1.0 / jnp.sqrt(deg)
    xw = jnp.dot(x_ref[...], w1_ref[...], precision=HIGHEST)
    y1_ref[...] = dinv * xw
    dinv_ref[...] = dinv


def _norm_relu(a0, a1, y, dinv, b, oh, oht, rc):
    h = dinv * (a0 + a1 + y) + b
    mean = jnp.dot(oht, h, precision=HIGHEST) * rc
    diff = h - jnp.dot(oh, mean, precision=HIGHEST)
    var = jnp.dot(oht, diff * diff, precision=HIGHEST) * rc
    hn = diff / jnp.sqrt(jnp.dot(oh, var, precision=HIGHEST) + EPS)
    return jnp.maximum(hn, 0.0)


def _onehots(batc, batr):
    oh = (batc == lax.broadcasted_iota(jnp.int32, (NP, G), 1)).astype(jnp.float32)
    oht = (batr == lax.broadcasted_iota(jnp.int32, (G, NP), 0)).astype(jnp.float32)
    rc = 1.0 / jnp.maximum(jnp.sum(oht, axis=1, keepdims=True), 1.0)
    return oh, oht, rc


def _tc_mid_body(a0_ref, a1_ref, y1_ref, dinv_ref, b1_ref, batc_ref, batr_ref,
                 w2_ref, y2_ref):
    oh, oht, rc = _onehots(batc_ref[...], batr_ref[...])
    h1 = _norm_relu(a0_ref[...], a1_ref[...], y1_ref[...], dinv_ref[...],
                    b1_ref[...], oh, oht, rc)
    y2_ref[...] = dinv_ref[...] * jnp.dot(h1, w2_ref[...], precision=HIGHEST)


def _tc_fin_body(a0_ref, a1_ref, y2_ref, dinv_ref, b2_ref, batc_ref, batr_ref,
                 wfc_ref, bfc_ref, out_ref):
    oh, oht, rc = _onehots(batc_ref[...], batr_ref[...])
    h2 = _norm_relu(a0_ref[...], a1_ref[...], y2_ref[...], dinv_ref[...],
                    b2_ref[...], oh, oht, rc)
    pooled = jnp.dot(oht, h2, precision=HIGHEST) * rc
    out_ref[...] = jnp.dot(pooled, wfc_ref[...], precision=HIGHEST) + bfc_ref[...]


_tc_pre = pl.pallas_call(
    _tc_pre_body,
    out_shape=[jax.ShapeDtypeStruct((NP, H), jnp.float32),
               jax.ShapeDtypeStruct((NP, 1), jnp.float32)])

_tc_mid = pl.pallas_call(
    _tc_mid_body,
    out_shape=jax.ShapeDtypeStruct((NP, H), jnp.float32))

_tc_fin = pl.pallas_call(
    _tc_fin_body,
    out_shape=jax.ShapeDtypeStruct((G, 128), jnp.float32))


def kernel(x, edge_index, batch, W1, b1, W2, b2, Wfc, bfc):
    src3 = edge_index[0].reshape(NW, CH, K)
    dst3 = edge_index[1].reshape(NW, CH, K)
    x_p = jnp.pad(x, ((0, NP - N), (0, 0)))
    batc = jnp.pad(batch.astype(jnp.int32), (0, NP - N),
                   constant_values=G).reshape(NP, 1)
    batr = batc.reshape(1, NP)
    wfc_p = jnp.pad(Wfc, ((0, 0), (0, 128 - C)))
    bfc_p = jnp.pad(bfc, (0, 128 - C)).reshape(1, 128)

    degp = _sc_deg(dst3)
    d0 = degp[0].reshape(NP, 1)
    d1 = degp[1].reshape(NP, 1)
    y1, dinv = _tc_pre(x_p, W1, d0, d1)

    ag1 = _sc_agg(y1, src3, dst3)
    y2 = _tc_mid(ag1[0], ag1[1], y1, dinv, b1.reshape(1, H), batc, batr, W2)

    ag2 = _sc_agg(y2, src3, dst3)
    outp = _tc_fin(ag2[0], ag2[1], y2, dinv, b2.reshape(1, H), batc, batr,
                   wfc_p, bfc_p)
    return outp[:, :C]


# speedup vs baseline: 18.1419x; 18.1419x over previous
import functools

import jax
import jax.numpy as jnp
from jax import lax
from jax.experimental import pallas as pl
from jax.experimental.pallas import tpu as pltpu
from jax.experimental.pallas import tpu_sc as plsc

N = 10000
E = 320000
D = 128
H = 128
C = 2
G = 64
EPS = 1e-5

NP = 10240
NC = 2
NS = 16
L = 16
NW = NC * NS
EW = E // NW
K = 80
CH = EW // K
SLAB = NP // NS
HIGHEST = jax.lax.Precision.HIGHEST

@functools.cache
def _mesh():
    return plsc.VectorSubcoreMesh(
        core_axis_name="c", subcore_axis_name="s",
        num_cores=NC, num_subcores=NS)


def _sc_deg_body(dst_hbm, out_hbm, didx_v, ones_v, zcol_v, acc_sh):
    core = lax.axis_index("c")
    sub = lax.axis_index("s")
    w = sub * NC + core

    for i in range(K // L):
        ones_v[pl.ds(i * L, L)] = jnp.ones((L,), jnp.float32)

    def _zero(i, _):
        zcol_v[pl.ds(i * L, L)] = jnp.zeros((L,), jnp.float32)
        return 0
    lax.fori_loop(0, SLAB // L, _zero, 0)
    pltpu.sync_copy(zcol_v, acc_sh.at[pl.ds(sub * SLAB, SLAB)])
    plsc.subcore_barrier()

    pltpu.sync_copy(dst_hbm.at[w], didx_v)

    def _chunk(j, _):
        pltpu.sync_copy(ones_v, acc_sh.at[didx_v.at[j]], add=True)
        return 0
    lax.fori_loop(0, CH, _chunk, 0)
    plsc.subcore_barrier()
    pltpu.sync_copy(acc_sh.at[pl.ds(sub * SLAB, SLAB)],
                    out_hbm.at[core, pl.ds(sub * SLAB, SLAB)])


@functools.cache
def _sc_deg():
    return pl.kernel(
        _sc_deg_body,
        out_type=jax.ShapeDtypeStruct((NC, NP), jnp.float32),
        mesh=_mesh(),
        scratch_types=[
            pltpu.VMEM((CH, K), jnp.int32),
            pltpu.VMEM((K,), jnp.float32),
            pltpu.VMEM((SLAB,), jnp.float32),
            pltpu.VMEM_SHARED((NP,), jnp.float32),
        ],
    )


def _sc_agg_body(y_hbm, src_hbm, dst_hbm, out_hbm, sidx_v, didx_v, rows_v,
                 acc_sh):
    core = lax.axis_index("c")
    sub = lax.axis_index("s")
    w = sub * NC + core

    def _zero(i, _):
        rows_v[i // 8, pl.ds((i % 8) * L, L)] = jnp.zeros((L,), jnp.float32)
        return 0
    lax.fori_loop(0, K * (H // L), _zero, 0)
    for t in range(SLAB // K):
        pltpu.sync_copy(rows_v, acc_sh.at[pl.ds(sub * SLAB + t * K, K)])
    plsc.subcore_barrier()

    pltpu.sync_copy(src_hbm.at[w], sidx_v)
    pltpu.sync_copy(dst_hbm.at[w], didx_v)

    def _chunk(j, _):
        pltpu.sync_copy(y_hbm.at[sidx_v.at[j]], rows_v)
        pltpu.sync_copy(rows_v, acc_sh.at[didx_v.at[j]], add=True)
        return 0
    lax.fori_loop(0, CH, _chunk, 0)
    plsc.subcore_barrier()
    pltpu.sync_copy(acc_sh.at[pl.ds(sub * SLAB, SLAB)],
                    out_hbm.at[core, pl.ds(sub * SLAB, SLAB)])


@functools.cache
def _sc_agg():
    return pl.kernel(
        _sc_agg_body,
        out_type=jax.ShapeDtypeStruct((NC, NP, H), jnp.float32),
        mesh=_mesh(),
        scratch_types=[
            pltpu.VMEM((CH, K), jnp.int32),
            pltpu.VMEM((CH, K), jnp.int32),
            pltpu.VMEM((K, H), jnp.float32),
            pltpu.VMEM_SHARED((NP, H), jnp.float32),
        ],
    )


def _tc_pre_body(x_ref, w1_ref, d0_ref, d1_ref, y1_ref, dinv_ref):
    deg = d0_ref[...] + d1_ref[...] + 1.0
    dinv = 1.0 / jnp.sqrt(deg)
    xw = jnp.dot(x_ref[...], w1_ref[...], precision=HIGHEST)
    y1_ref[...] = dinv * xw
    dinv_ref[...] = dinv


def _tc_stats_body(a0_ref, a1_ref, y_ref, dinv_ref, b_ref, batr_ref,
                   h_ref, scale_ref, shift_ref):
    h = dinv_ref[...] * (a0_ref[...] + a1_ref[...] + y_ref[...]) + b_ref[...]
    oht = (batr_ref[...] ==
           lax.broadcasted_iota(jnp.int32, (G, NP), 0)).astype(jnp.float32)
    rc = 1.0 / jnp.maximum(jnp.sum(oht, axis=1, keepdims=True), 1.0)
    mean = jnp.dot(oht, h, precision=HIGHEST) * rc
    ex2 = jnp.dot(oht, h * h, precision=HIGHEST) * rc
    var = ex2 - mean * mean
    scale = 1.0 / jnp.sqrt(var + EPS)
    h_ref[...] = h
    scale_ref[...] = scale
    shift_ref[...] = -mean * scale


def _tc_apply_body(h_ref, scale_ref, shift_ref, batc_ref, dinv_ref, w2_ref,
                   y2_ref):
    oh = (batc_ref[...] ==
          lax.broadcasted_iota(jnp.int32, (NP, G), 1)).astype(jnp.float32)
    hn = (h_ref[...] * jnp.dot(oh, scale_ref[...], precision=HIGHEST)
          + jnp.dot(oh, shift_ref[...], precision=HIGHEST))
    h1 = jnp.maximum(hn, 0.0)
    y2_ref[...] = dinv_ref[...] * jnp.dot(h1, w2_ref[...], precision=HIGHEST)


def _tc_pool_body(h_ref, scale_ref, shift_ref, batc_ref, batr_ref, wfc_ref,
                  bfc_ref, out_ref):
    oh = (batc_ref[...] ==
          lax.broadcasted_iota(jnp.int32, (NP, G), 1)).astype(jnp.float32)
    oht = (batr_ref[...] ==
           lax.broadcasted_iota(jnp.int32, (G, NP), 0)).astype(jnp.float32)
    rc = 1.0 / jnp.maximum(jnp.sum(oht, axis=1, keepdims=True), 1.0)
    hn = (h_ref[...] * jnp.dot(oh, scale_ref[...], precision=HIGHEST)
          + jnp.dot(oh, shift_ref[...], precision=HIGHEST))
    h2 = jnp.maximum(hn, 0.0)
    pooled = jnp.dot(oht, h2, precision=HIGHEST) * rc
    out_ref[...] = jnp.dot(pooled, wfc_ref[...], precision=HIGHEST) + bfc_ref[...]


_tc_pre = pl.pallas_call(
    _tc_pre_body,
    out_shape=[jax.ShapeDtypeStruct((NP, H), jnp.float32),
               jax.ShapeDtypeStruct((NP, 1), jnp.float32)])

_tc_stats = pl.pallas_call(
    _tc_stats_body,
    out_shape=[jax.ShapeDtypeStruct((NP, H), jnp.float32),
               jax.ShapeDtypeStruct((G, H), jnp.float32),
               jax.ShapeDtypeStruct((G, H), jnp.float32)])

_tc_apply = pl.pallas_call(
    _tc_apply_body,
    out_shape=jax.ShapeDtypeStruct((NP, H), jnp.float32))

_tc_pool = pl.pallas_call(
    _tc_pool_body,
    out_shape=jax.ShapeDtypeStruct((G, 128), jnp.float32))


def kernel(x, edge_index, batch, W1, b1, W2, b2, Wfc, bfc):
    src3 = edge_index[0].reshape(NW, CH, K)
    dst3 = edge_index[1].reshape(NW, CH, K)
    x_p = jnp.pad(x, ((0, NP - N), (0, 0)))
    batc = jnp.pad(batch.astype(jnp.int32), (0, NP - N),
                   constant_values=G).reshape(NP, 1)
    batr = batc.reshape(1, NP)
    wfc_p = jnp.pad(Wfc, ((0, 0), (0, 128 - C)))
    bfc_p = jnp.pad(bfc, (0, 128 - C)).reshape(1, 128)

    degp = _sc_deg()(dst3)
    d0 = degp[0].reshape(NP, 1)
    d1 = degp[1].reshape(NP, 1)
    y1, dinv = _tc_pre(x_p, W1, d0, d1)

    ag1 = _sc_agg()(y1, src3, dst3)
    h1, sc1, sh1 = _tc_stats(ag1[0], ag1[1], y1, dinv, b1.reshape(1, H), batr)
    y2 = _tc_apply(h1, sc1, sh1, batc, dinv, W2)

    ag2 = _sc_agg()(y2, src3, dst3)
    h2, sc2, sh2 = _tc_stats(ag2[0], ag2[1], y2, dinv, b2.reshape(1, H), batr)
    outp = _tc_pool(h2, sc2, sh2, batc, batr, wfc_p, bfc_p)
    return outp[:, :C]
